# baseline (speedup 1.0000x reference)
import functools

import jax
import jax.numpy as jnp
from jax import lax
from jax.experimental import pallas as pl
from jax.experimental.pallas import tpu as pltpu
from jax.experimental.pallas import tpu_sc as plsc

B, H, F, D, V = 4096, 50, 26, 32, 100000
NC, NS, L = 2, 16, 16
NW = NC * NS
BPW = B // NW
NCH = BPW // 2
EPH = H * F
EPHP = 1312
NT16 = EPHP // 16
GSZ = [128] * 10 + [32]
GOFF = [0, 128, 256, 384, 512, 640, 768, 896, 1024, 1152, 1280]

_mesh = plsc.VectorSubcoreMesh(
    core_axis_name="c", subcore_axis_name="s", num_cores=NC, num_subcores=NS
)


@functools.partial(
    pl.kernel,
    out_type=jax.ShapeDtypeStruct((B, H, D), jnp.float32),
    mesh=_mesh,
    scratch_types=[
        pltpu.VMEM((2, H, F), jnp.int32),
        pltpu.VMEM((2, 2, EPHP), jnp.int32),
        pltpu.VMEM((EPHP, D), jnp.float32),
        pltpu.VMEM((EPHP, D), jnp.float32),
        pltpu.VMEM((H, D), jnp.float32),
        pltpu.SemaphoreType.DMA,
        pltpu.SemaphoreType.DMA,
    ],
    compiler_params=pltpu.CompilerParams(use_tc_tiling_on_sc=False),
)
def _embed_sum(nf_hbm, w_hbm, out_hbm, nfc_v, fidx_v, rows0_v, rows1_v,
               outb_v, sem0, sem1):
    cid = lax.axis_index("c")
    sid = lax.axis_index("s")
    wid = sid * NC + cid
    wtab = w_hbm.at[0]
    rows = (rows0_v, rows1_v)
    sems = (sem0, sem1)

    def load_chunk(c):
        cb = wid * BPW + c * 2
        pltpu.sync_copy(nf_hbm.at[pl.ds(cb, 2)], nfc_v)
        par = lax.rem(c, 2)
        for h in range(2):
            off0 = lax.iota(jnp.int32, 16) * V
            off1 = off0 + 10 * V
            zero = jnp.zeros((16,), jnp.int32)
            fidx_v[par, h, pl.ds(EPHP - 16, 16)] = zero

            def build(p, carry):
                e0 = nfc_v[h, p, pl.ds(0, 16)]
                fidx_v[par, h, pl.ds(p * F, 16)] = e0 + off0
                e1 = nfc_v[h, p, pl.ds(10, 16)]
                fidx_v[par, h, pl.ds(p * F + 10, 16)] = e1 + off1
                return carry

            lax.fori_loop(0, H, build, 0)

    def fire(c, h, buf):
        par = lax.rem(c, 2)
        for j, g in enumerate(GSZ):
            pltpu.async_copy(
                wtab.at[fidx_v.at[par, h, pl.ds(GOFF[j], g)]],
                rows[buf].at[pl.ds(GOFF[j], g)],
                sems[buf],
            )

    def drain_acc(c, h, buf):
        pltpu.make_async_copy(wtab.at[pl.ds(0, EPHP)], rows[buf], sems[buf]).wait()
        rv = rows[buf]

        def acc(p, carry):
            base = p * F
            a0 = rv[base, pl.ds(0, 16)]
            a1 = rv[base, pl.ds(16, 16)]
            for i in range(1, F):
                a0 = a0 + rv[base + i, pl.ds(0, 16)]
                a1 = a1 + rv[base + i, pl.ds(16, 16)]
            outb_v[p, pl.ds(0, 16)] = a0
            outb_v[p, pl.ds(16, 16)] = a1
            return carry

        lax.fori_loop(0, H, acc, 0)
        pltpu.sync_copy(outb_v, out_hbm.at[wid * BPW + c * 2 + h])

    load_chunk(0)
    fire(0, 0, 0)
    fire(0, 1, 1)

    def body(k, carry):
        drain_acc(k, 0, 0)

        @pl.when(k < NCH - 1)
        def _prefetch_even():
            load_chunk(k + 1)
            fire(k + 1, 0, 0)

        drain_acc(k, 1, 1)

        @pl.when(k < NCH - 1)
        def _prefetch_odd():
            fire(k + 1, 1, 1)

        return carry

    lax.fori_loop(0, NCH, body, 0)


def kernel(node_feature, W):
    return _embed_sum(node_feature, W)


# speedup vs baseline: 1.3187x; 1.3187x over previous
import functools

import jax
import jax.numpy as jnp
import numpy as np
from jax import lax
from jax.experimental import pallas as pl
from jax.experimental.pallas import tpu as pltpu
from jax.experimental.pallas import tpu_sc as plsc

B, H, F, D, V = 4096, 50, 26, 32, 100000
NC, NS = 2, 16
NW = NC * NS
P = B * H
PPW = P // NW
CP = 64
NCH = PPW // CP
E = CP * F
NG = E // 128
OR = CP // 4
ORPW = PPW // 4

_OFF_NP = (np.arange(E, dtype=np.int32) % F) * V

_mesh = plsc.VectorSubcoreMesh(
    core_axis_name="c", subcore_axis_name="s", num_cores=NC, num_subcores=NS
)


@functools.partial(
    pl.kernel,
    out_type=jax.ShapeDtypeStruct((P // 4, 128), jnp.float32),
    mesh=_mesh,
    scratch_types=[
        pltpu.VMEM((E,), jnp.int32),
        pltpu.VMEM((2, E), jnp.int32),
        pltpu.VMEM((E,), jnp.int32),
        pltpu.VMEM((E, D), jnp.float32),
        pltpu.VMEM((E, D), jnp.float32),
        pltpu.VMEM((OR, 128), jnp.float32),
        pltpu.SemaphoreType.DMA,
        pltpu.SemaphoreType.DMA,
    ],
    compiler_params=pltpu.CompilerParams(use_tc_tiling_on_sc=False),
)
def _embed_sum(nf_hbm, w_hbm, off_hbm, out_hbm, idxc_v, fidx_v, off_v,
               rows0_v, rows1_v, outb_v, sem0, sem1):
    cid = lax.axis_index("c")
    sid = lax.axis_index("s")
    wid = sid * NC + cid
    wtab = w_hbm.at[0]
    rows = (rows0_v, rows1_v)
    sems = (sem0, sem1)

    pltpu.sync_copy(off_hbm, off_v)

    def prep_fire(c, buf):
        pltpu.sync_copy(nf_hbm.at[pl.ds((wid * PPW + c * CP) * F, E)], idxc_v)

        def add16(g, carry):
            fidx_v[buf, pl.ds(g * 16, 16)] = (
                idxc_v[pl.ds(g * 16, 16)] + off_v[pl.ds(g * 16, 16)]
            )
            return carry

        lax.fori_loop(0, E // 16, add16, 0)
        for j in range(NG):
            pltpu.async_copy(
                wtab.at[fidx_v.at[buf, pl.ds(j * 128, 128)]],
                rows[buf].at[pl.ds(j * 128, 128)],
                sems[buf],
            )

    def drain_acc(c, buf):
        pltpu.make_async_copy(wtab.at[pl.ds(0, E)], rows[buf], sems[buf]).wait()
        rv = rows[buf]

        def acc(r, carry):
            for q in range(4):
                base = r * (4 * F) + q * F
                a0 = rv[base, pl.ds(0, 16)]
                a1 = rv[base, pl.ds(16, 16)]
                for i in range(1, F):
                    a0 = a0 + rv[base + i, pl.ds(0, 16)]
                    a1 = a1 + rv[base + i, pl.ds(16, 16)]
                outb_v[r, pl.ds(q * 32, 16)] = a0
                outb_v[r, pl.ds(q * 32 + 16, 16)] = a1
            return carry

        lax.fori_loop(0, OR, acc, 0)
        pltpu.sync_copy(outb_v, out_hbm.at[pl.ds(wid * ORPW + c * OR, OR)])

    prep_fire(0, 0)

    def body(k, carry):
        c0 = k * 2
        prep_fire(c0 + 1, 1)
        drain_acc(c0, 0)

        @pl.when(c0 + 2 < NCH)
        def _prefetch_even():
            prep_fire(c0 + 2, 0)

        drain_acc(c0 + 1, 1)
        return carry

    lax.fori_loop(0, NCH // 2, body, 0)


def kernel(node_feature, W):
    nf_flat = node_feature.reshape(-1)
    off = jnp.asarray(_OFF_NP)
    out = _embed_sum(nf_flat, W, off)
    return out.reshape(B, H, D)


# speedup vs baseline: 1.3718x; 1.0403x over previous
import functools

import jax
import jax.numpy as jnp
import numpy as np
from jax import lax
from jax.experimental import pallas as pl
from jax.experimental.pallas import tpu as pltpu
from jax.experimental.pallas import tpu_sc as plsc

B, H, F, D, V = 4096, 50, 26, 32, 100000
NC, NS = 2, 16
NW = NC * NS
P = B * H
PPW = P // NW
CP = 64
NCH = PPW // CP
E = CP * F
NG = E // 128
OR = CP // 4
ORPW = PPW // 4

_OFF_NP = (np.arange(E, dtype=np.int32) % F) * V

_mesh = plsc.VectorSubcoreMesh(
    core_axis_name="c", subcore_axis_name="s", num_cores=NC, num_subcores=NS
)


@functools.partial(
    pl.kernel,
    out_type=jax.ShapeDtypeStruct((P // 4, 128), jnp.float32),
    mesh=_mesh,
    scratch_types=[
        pltpu.VMEM((2, E), jnp.int32),
        pltpu.VMEM((2, E), jnp.int32),
        pltpu.VMEM((E,), jnp.int32),
        pltpu.VMEM((E, D), jnp.float32),
        pltpu.VMEM((E, D), jnp.float32),
        pltpu.VMEM((OR, 128), jnp.float32),
        pltpu.SemaphoreType.DMA,
        pltpu.SemaphoreType.DMA,
        pltpu.SemaphoreType.DMA,
        pltpu.SemaphoreType.DMA,
    ],
    compiler_params=pltpu.CompilerParams(use_tc_tiling_on_sc=False),
)
def _embed_sum(nf_hbm, w_hbm, off_hbm, out_hbm, idxc_v, fidx_v, off_v,
               rows0_v, rows1_v, outb_v, sem0, sem1, isem0, isem1):
    cid = lax.axis_index("c")
    sid = lax.axis_index("s")
    wid = sid * NC + cid
    wtab = w_hbm.at[0]
    rows = (rows0_v, rows1_v)
    sems = (sem0, sem1)
    isems = (isem0, isem1)

    pltpu.sync_copy(off_hbm, off_v)

    def idx_fetch(c, buf):
        pltpu.async_copy(
            nf_hbm.at[pl.ds((wid * PPW + c * CP) * F, E)],
            idxc_v.at[buf],
            isems[buf],
        )

    def prep_fire(c, buf):
        pltpu.make_async_copy(
            nf_hbm.at[pl.ds(0, E)], idxc_v.at[buf], isems[buf]
        ).wait()

        def add16(g, carry):
            fidx_v[buf, pl.ds(g * 16, 16)] = (
                idxc_v[buf, pl.ds(g * 16, 16)] + off_v[pl.ds(g * 16, 16)]
            )
            return carry

        lax.fori_loop(0, E // 16, add16, 0)

        @pl.when(c + 2 < NCH)
        def _next_idx():
            idx_fetch(c + 2, buf)

        for j in range(NG):
            pltpu.async_copy(
                wtab.at[fidx_v.at[buf, pl.ds(j * 128, 128)]],
                rows[buf].at[pl.ds(j * 128, 128)],
                sems[buf],
            )

    def drain_acc(c, buf):
        pltpu.make_async_copy(wtab.at[pl.ds(0, E)], rows[buf], sems[buf]).wait()
        rv = rows[buf]

        def acc(r, carry):
            for q in range(4):
                base = r * (4 * F) + q * F
                a0 = rv[base, pl.ds(0, 16)]
                a1 = rv[base, pl.ds(16, 16)]
                for i in range(1, F):
                    a0 = a0 + rv[base + i, pl.ds(0, 16)]
                    a1 = a1 + rv[base + i, pl.ds(16, 16)]
                outb_v[r, pl.ds(q * 32, 16)] = a0
                outb_v[r, pl.ds(q * 32 + 16, 16)] = a1
            return carry

        lax.fori_loop(0, OR, acc, 0)
        pltpu.sync_copy(outb_v, out_hbm.at[pl.ds(wid * ORPW + c * OR, OR)])

    idx_fetch(0, 0)
    idx_fetch(1, 1)
    prep_fire(0, 0)

    def body(k, carry):
        c0 = k * 2
        prep_fire(c0 + 1, 1)
        drain_acc(c0, 0)

        @pl.when(c0 + 2 < NCH)
        def _prefetch_even():
            prep_fire(c0 + 2, 0)

        drain_acc(c0 + 1, 1)
        return carry

    lax.fori_loop(0, NCH // 2, body, 0)


def kernel(node_feature, W):
    nf_flat = node_feature.reshape(-1)
    off = jnp.asarray(_OFF_NP)
    out = _embed_sum(nf_flat, W, off)
    return out.reshape(B, H, D)
